# Initial kernel scaffold; baseline (speedup 1.0000x reference)
import jax
import jax.numpy as jnp
from jax.experimental import pallas as pl


def kernel(x, y, z, labels):
    raise NotImplementedError("write your pallas kernel here")


# speedup vs baseline: 3.5048x; 3.5048x over previous
import functools

import jax
import jax.numpy as jnp
from jax import lax
from jax.experimental import pallas as pl
from jax.experimental.pallas import tpu as pltpu
from jax.experimental.pallas import tpu_sc as plsc

S = 512
CLASSES = 64
SHIFT = S // 2
N = 1048576
F = S * S * CLASSES

NT = 16
PPT = N // NT
CELLS_PT = F // NT
ZB = 32768
NZ = CELLS_PT // ZB
CHUNK = 16384
NCH = PPT // CHUNK
ROW = 128
ROWS = CHUNK // ROW

MAGIC_F = 12582912.0
MAGIC_I = 0x4B400000
R_F = 0.05

_mesh = plsc.VectorSubcoreMesh(
    core_axis_name="c", subcore_axis_name="s", num_cores=1
)


@functools.partial(
    pl.kernel,
    out_type=jax.ShapeDtypeStruct((F,), jnp.float32),
    mesh=_mesh,
    compiler_params=pltpu.CompilerParams(needs_layout_passes=False),
    scratch_types=[
        pltpu.VMEM((ZB,), jnp.float32),
        pltpu.VMEM((CHUNK,), jnp.float32),
        pltpu.VMEM((CHUNK,), jnp.float32),
        pltpu.VMEM((CHUNK,), jnp.int32),
        pltpu.VMEM((ROWS, ROW), jnp.int32),
        pltpu.VMEM((ROWS, ROW), jnp.float32),
        pltpu.SemaphoreType.DMA,
        pltpu.SemaphoreType.DMA,
    ],
)
def _graph_map_kernel(x_hbm, z_hbm, lab_hbm, out_hbm,
                      zbuf, xb, zb, lb, idxb, valb, zsem, ssem):
    tid = lax.axis_index("s")

    def _zfill(i, carry):
        zbuf[pl.ds(i * 16, 16)] = jnp.zeros((16,), jnp.float32)
        return carry

    lax.fori_loop(0, ZB // 16, _zfill, 0)

    zbase = tid * CELLS_PT
    zero_copies = [
        pltpu.async_copy(zbuf, out_hbm.at[pl.ds(zbase + j * ZB, ZB)], zsem)
        for j in range(NZ)
    ]
    for c in zero_copies:
        c.wait()

    plsc.subcore_barrier()

    pbase = tid * PPT
    for ch in range(NCH):
        cbase = pbase + ch * CHUNK
        pltpu.sync_copy(x_hbm.at[pl.ds(cbase, CHUNK)], xb)
        pltpu.sync_copy(z_hbm.at[pl.ds(cbase, CHUNK)], zb)
        pltpu.sync_copy(lab_hbm.at[pl.ds(cbase, CHUNK)], lb)

        def _row(k, carry):
            def _lanes(i, c2):
                o = k * ROW + i * 16
                xv = xb[pl.ds(o, 16)]
                zv = zb[pl.ds(o, 16)]
                lv = lb[pl.ds(o, 16)]
                xi = plsc.bitcast(xv / R_F + MAGIC_F, jnp.int32) - (
                    MAGIC_I - SHIFT)
                zi = plsc.bitcast(zv / R_F + MAGIC_F, jnp.int32) - (
                    MAGIC_I - SHIFT)
                xi = jnp.minimum(jnp.maximum(xi, 0), S - 1)
                zi = jnp.minimum(jnp.maximum(zi, 0), S - 1)
                flat = (xi << 15) + (zi << 6) + lv
                idxb[k, pl.ds(i * 16, 16)] = flat
                valb[k, pl.ds(i * 16, 16)] = lv.astype(jnp.float32)
                return c2

            lax.fori_loop(0, ROW // 16, _lanes, 0)
            pltpu.async_copy(valb.at[k], out_hbm.at[idxb.at[k]], ssem)
            return carry

        lax.fori_loop(0, ROWS, _row, 0)

        pltpu.make_async_copy(x_hbm.at[pl.ds(0, CHUNK)], xb, ssem).wait()


def kernel(x, y, z, labels):
    del y
    flat = _graph_map_kernel(x, z, labels)
    return flat.reshape(S, S, CLASSES)
